# baseline (speedup 1.0000x reference)
import functools

import jax
import jax.numpy as jnp
from jax import lax
from jax.experimental import pallas as pl
from jax.experimental.pallas import tpu as pltpu
from jax.experimental.pallas import tpu_sc as plsc

N = 10000
D = 128
E = 320000
NC = 2
NS = 16
CHUNK = 64
NCHUNK = 160
NHALF = 4
HCH = NCHUNK // NHALF
NBUF = 4
LAG = 3
NGRP = HCH // NBUF
E_PAD = NC * NS * NCHUNK * CHUNK
NPAD = 10240
ROWS_PT = NPAD // NS
DUMMY_DST = NPAD - 1


def _seg_body(with_counts, x_hbm, src_hbm, dst_hbm, psum_hbm, *rest):
    if with_counts:
        (pcnt_hbm, acc_sh, cnt_sh, src_v, dst_v, rows, gsems, ssems, ones_v,
         czv) = rest
    else:
        (acc_sh, src_v, dst_v, rows, gsems, ssems) = rest
        ones_v = czv = None
    c = lax.axis_index("c")
    s = lax.axis_index("s")

    zero16 = jnp.zeros((16,), jnp.float32)

    def _zrows(i, _):
        rows[0][i // 8, pl.ds((i % 8) * 16, 16)] = zero16
        return 0
    lax.fori_loop(0, CHUNK * 8, _zrows, 0)

    if with_counts:
        one16 = jnp.ones((16,), jnp.float32)

        def _ones(i, _):
            ones_v[pl.ds(i * 16, 16)] = one16
            return 0
        lax.fori_loop(0, CHUNK // 16, _ones, 0)

        def _zc(i, _):
            czv[pl.ds(i * 16, 16)] = zero16
            return 0
        lax.fori_loop(0, ROWS_PT // 16, _zc, 0)

    def _zslab(k, _):
        pltpu.sync_copy(rows[0],
                        acc_sh.at[pl.ds(s * ROWS_PT + k * CHUNK, CHUNK)])
        return 0
    lax.fori_loop(0, ROWS_PT // CHUNK, _zslab, 0)
    if with_counts:
        pltpu.sync_copy(czv, cnt_sh.at[pl.ds(s * ROWS_PT, ROWS_PT)])
    plsc.subcore_barrier()

    def _start_gather(j, b):
        pltpu.async_copy(x_hbm.at[src_v.at[j]], rows[b], gsems[b])

    def _wait_gather(j, b):
        pltpu.make_async_copy(x_hbm.at[src_v.at[j]], rows[b],
                              gsems[b]).wait()

    def _start_scat(j, b):
        pltpu.async_copy(rows[b], acc_sh.at[dst_v.at[j]], ssems[b], add=True)

    def _wait_scat(j, b):
        pltpu.make_async_copy(rows[b], acc_sh.at[dst_v.at[j]],
                              ssems[b]).wait()

    def _counts(j):
        if with_counts:
            pltpu.sync_copy(ones_v, cnt_sh.at[dst_v.at[j]], add=True)

    def _half(h, _):
        pltpu.sync_copy(src_hbm.at[c, s, pl.ds(h * HCH, HCH)], src_v)
        pltpu.sync_copy(dst_hbm.at[c, s, pl.ds(h * HCH, HCH)], dst_v)
        for b in range(LAG):
            _start_gather(b, b)

        def _grp_body(gi, carry):
            for b in range(NBUF):
                t = gi * NBUF + b
                _wait_gather(t, b)
                _start_scat(t, b)
                _counts(t)
                pb = (b + LAG) % NBUF
                nxt = t + LAG
                if b < NBUF - LAG:
                    @pl.when(gi > 0)
                    def _():
                        _wait_scat(t - 1, pb)
                        _start_gather(nxt, pb)

                    @pl.when(gi == 0)
                    def _():
                        _start_gather(nxt, pb)
                else:
                    @pl.when(nxt < HCH)
                    def _():
                        _wait_scat(t - 1, pb)
                        _start_gather(nxt, pb)
            return carry
        lax.fori_loop(0, NGRP, _grp_body, 0)
        for b in range(NBUF):
            _wait_scat(HCH - NBUF + b, (HCH - NBUF + b) % NBUF)
        return 0
    lax.fori_loop(0, NHALF, _half, 0)
    plsc.subcore_barrier()

    pltpu.sync_copy(acc_sh.at[pl.ds(s * ROWS_PT, ROWS_PT)],
                    psum_hbm.at[c, pl.ds(s * ROWS_PT, ROWS_PT)])
    if with_counts:
        pltpu.sync_copy(cnt_sh.at[pl.ds(s * ROWS_PT, ROWS_PT)],
                        pcnt_hbm.at[c, pl.ds(s * ROWS_PT, ROWS_PT)])


def _make_seg_sum(with_counts):
    out_type = [jax.ShapeDtypeStruct((NC, NPAD, D), jnp.float32)]
    scratch = [pltpu.VMEM_SHARED((NPAD, D), jnp.float32)]
    if with_counts:
        out_type.append(jax.ShapeDtypeStruct((NC, NPAD), jnp.float32))
        scratch.append(pltpu.VMEM_SHARED((NPAD,), jnp.float32))
    scratch += [
        pltpu.VMEM((HCH, CHUNK), jnp.int32),
        pltpu.VMEM((HCH, CHUNK), jnp.int32),
        [pltpu.VMEM((CHUNK, D), jnp.float32) for _ in range(NBUF)],
        [pltpu.SemaphoreType.DMA for _ in range(NBUF)],
        [pltpu.SemaphoreType.DMA for _ in range(NBUF)],
    ]
    if with_counts:
        scratch += [
            pltpu.VMEM((CHUNK,), jnp.float32),
            pltpu.VMEM((ROWS_PT,), jnp.float32),
        ]
    return pl.kernel(
        functools.partial(_seg_body, with_counts),
        out_type=tuple(out_type),
        mesh=plsc.VectorSubcoreMesh(core_axis_name="c", subcore_axis_name="s",
                                    num_cores=NC, num_subcores=NS),
        scratch_types=scratch,
    )


_seg_sum_cnt = _make_seg_sum(True)
_seg_sum = _make_seg_sum(False)


BLK = 1280


def _layer_body(relu, p_ref, c_ref, x_ref, wl_ref, bl_ref, wr_ref, o_ref):
    i = pl.program_id(0)
    cnt = c_ref[0, pl.ds(i * BLK, BLK)] + c_ref[1, pl.ds(i * BLK, BLK)]
    recip = 1.0 / jnp.maximum(cnt, 1.0)
    mean = (p_ref[0] + p_ref[1]) * recip[:, None]
    t = (jnp.dot(mean, wl_ref[...], preferred_element_type=jnp.float32)
         + jnp.dot(x_ref[...], wr_ref[...], preferred_element_type=jnp.float32)
         + bl_ref[...])
    o_ref[...] = jnp.maximum(t, 0.0) if relu else t


def _make_layer(relu):
    return pl.pallas_call(
        functools.partial(_layer_body, relu),
        grid=(NPAD // BLK,),
        in_specs=[
            pl.BlockSpec((NC, BLK, D), lambda i: (0, i, 0)),
            pl.BlockSpec((NC, NPAD), lambda i: (0, 0)),
            pl.BlockSpec((BLK, D), lambda i: (i, 0)),
            pl.BlockSpec((D, D), lambda i: (0, 0)),
            pl.BlockSpec((1, D), lambda i: (0, 0)),
            pl.BlockSpec((D, D), lambda i: (0, 0)),
        ],
        out_specs=pl.BlockSpec((BLK, D), lambda i: (i, 0)),
        out_shape=jax.ShapeDtypeStruct((NPAD, D), jnp.float32),
    )


_layer_relu = _make_layer(True)
_layer_lin = _make_layer(False)


def kernel(x, edge_index, Wl1, bl1, Wr1, Wl2, bl2, Wr2):
    pad = E_PAD - E
    src = jnp.concatenate(
        [edge_index[0], jnp.zeros((pad,), jnp.int32)]).reshape(
            NC, NS, NCHUNK, CHUNK)
    dst = jnp.concatenate(
        [edge_index[1], jnp.full((pad,), DUMMY_DST, jnp.int32)]).reshape(
            NC, NS, NCHUNK, CHUNK)
    p1, c1 = _seg_sum_cnt(x, src, dst)
    h = _layer_relu(p1, c1, x, Wl1, bl1.reshape(1, D), Wr1)
    (p2,) = _seg_sum(h, src, dst)
    out = _layer_lin(p2, c1, h, Wl2, bl2.reshape(1, D), Wr2)
    return out[:N]


# speedup vs baseline: 1.1704x; 1.1704x over previous
import functools

import jax
import jax.numpy as jnp
from jax import lax
from jax.experimental import pallas as pl
from jax.experimental.pallas import tpu as pltpu
from jax.experimental.pallas import tpu_sc as plsc

N = 10000
D = 128
E = 320000
NC = 2
NS = 16
CHUNK = 128
NCHUNK0 = 128
NCHUNK1 = 32
NCHUNK_TOT = NCHUNK0 + NCHUNK1
HCH = 16
NBUF = 2
LAG = 1
NGRP = HCH // NBUF
E_PAD = NS * NCHUNK_TOT * CHUNK
NPAD = 10240
ROWS_PT = NPAD // NS
DUMMY_LO = N


def _seg_body(with_counts, x_hbm, src_hbm, dst_hbm, psum_hbm, *rest):
    if with_counts:
        (pcnt_hbm, acc_sh, cnt_sh, src_v, dst_v, rows, gsems, ssems, ones_v,
         czv) = rest
    else:
        (acc_sh, src_v, dst_v, rows, gsems, ssems) = rest
        ones_v = czv = None
    c = lax.axis_index("c")
    s = lax.axis_index("s")

    zero16 = jnp.zeros((16,), jnp.float32)

    def _zrows(i, _):
        rows[0][i // 8, pl.ds((i % 8) * 16, 16)] = zero16
        return 0
    lax.fori_loop(0, CHUNK * 8, _zrows, 0)

    if with_counts:
        one16 = jnp.ones((16,), jnp.float32)

        def _ones(i, _):
            ones_v[pl.ds(i * 16, 16)] = one16
            return 0
        lax.fori_loop(0, CHUNK // 16, _ones, 0)

        def _zc(i, _):
            czv[pl.ds(i * 16, 16)] = zero16
            return 0
        lax.fori_loop(0, ROWS_PT // 16, _zc, 0)

    def _zslab(k, _):
        pltpu.sync_copy(rows[0],
                        acc_sh.at[pl.ds(s * ROWS_PT + k * CHUNK, CHUNK)])
        return 0
    lax.fori_loop(0, ROWS_PT // CHUNK, _zslab, 0)
    if with_counts:
        pltpu.sync_copy(czv, cnt_sh.at[pl.ds(s * ROWS_PT, ROWS_PT)])
    plsc.subcore_barrier()

    def _start_gather(j, b):
        pltpu.async_copy(x_hbm.at[src_v.at[j]], rows[b], gsems[b])

    def _wait_gather(j, b):
        pltpu.make_async_copy(x_hbm.at[src_v.at[j]], rows[b],
                              gsems[b]).wait()

    def _start_scat(j, b):
        pltpu.async_copy(rows[b], acc_sh.at[dst_v.at[j]], ssems[b], add=True)

    def _wait_scat(j, b):
        pltpu.make_async_copy(rows[b], acc_sh.at[dst_v.at[j]],
                              ssems[b]).wait()

    def _counts(j):
        if with_counts:
            pltpu.sync_copy(ones_v, cnt_sh.at[dst_v.at[j]], add=True)

    npieces = jnp.where(c == 0, NCHUNK0 // HCH, NCHUNK1 // HCH)

    def _piece(p, _):
        base = pl.multiple_of(c * NCHUNK0 + p * HCH, 8)
        pltpu.sync_copy(src_hbm.at[s, pl.ds(base, HCH)], src_v)
        pltpu.sync_copy(dst_hbm.at[s, pl.ds(base, HCH)], dst_v)
        for b in range(LAG):
            _start_gather(b, b)

        def _grp_body(gi, carry):
            for b in range(NBUF):
                t = gi * NBUF + b
                _wait_gather(t, b)
                _start_scat(t, b)
                _counts(t)
                pb = (b + LAG) % NBUF
                nxt = t + LAG
                if b < NBUF - LAG:
                    @pl.when(gi > 0)
                    def _():
                        _wait_scat(t - 1, pb)
                        _start_gather(nxt, pb)

                    @pl.when(gi == 0)
                    def _():
                        _start_gather(nxt, pb)
                else:
                    @pl.when(nxt < HCH)
                    def _():
                        _wait_scat(t - 1, pb)
                        _start_gather(nxt, pb)
            return carry
        lax.fori_loop(0, NGRP, _grp_body, 0)
        for b in range(NBUF):
            _wait_scat(HCH - NBUF + b, (HCH - NBUF + b) % NBUF)
        return 0
    lax.fori_loop(0, npieces, _piece, 0)
    plsc.subcore_barrier()

    pltpu.sync_copy(acc_sh.at[pl.ds(s * ROWS_PT, ROWS_PT)],
                    psum_hbm.at[c, pl.ds(s * ROWS_PT, ROWS_PT)])
    if with_counts:
        pltpu.sync_copy(cnt_sh.at[pl.ds(s * ROWS_PT, ROWS_PT)],
                        pcnt_hbm.at[c, pl.ds(s * ROWS_PT, ROWS_PT)])


def _make_seg_sum(with_counts):
    out_type = [jax.ShapeDtypeStruct((NC, NPAD, D), jnp.float32)]
    scratch = [pltpu.VMEM_SHARED((NPAD, D), jnp.float32)]
    if with_counts:
        out_type.append(jax.ShapeDtypeStruct((NC, NPAD), jnp.float32))
        scratch.append(pltpu.VMEM_SHARED((NPAD,), jnp.float32))
    scratch += [
        pltpu.VMEM((HCH, CHUNK), jnp.int32),
        pltpu.VMEM((HCH, CHUNK), jnp.int32),
        [pltpu.VMEM((CHUNK, D), jnp.float32) for _ in range(NBUF)],
        [pltpu.SemaphoreType.DMA for _ in range(NBUF)],
        [pltpu.SemaphoreType.DMA for _ in range(NBUF)],
    ]
    if with_counts:
        scratch += [
            pltpu.VMEM((CHUNK,), jnp.float32),
            pltpu.VMEM((ROWS_PT,), jnp.float32),
        ]
    return pl.kernel(
        functools.partial(_seg_body, with_counts),
        out_type=tuple(out_type),
        mesh=plsc.VectorSubcoreMesh(core_axis_name="c", subcore_axis_name="s",
                                    num_cores=NC, num_subcores=NS),
        scratch_types=scratch,
    )


_seg_sum_cnt = _make_seg_sum(True)
_seg_sum = _make_seg_sum(False)


BLK = 1280


def _layer_body(relu, p_ref, c_ref, x_ref, wl_ref, bl_ref, wr_ref, o_ref):
    i = pl.program_id(0)
    cnt = c_ref[0, pl.ds(i * BLK, BLK)] + c_ref[1, pl.ds(i * BLK, BLK)]
    recip = 1.0 / jnp.maximum(cnt, 1.0)
    mean = (p_ref[0] + p_ref[1]) * recip[:, None]
    t = (jnp.dot(mean, wl_ref[...], preferred_element_type=jnp.float32)
         + jnp.dot(x_ref[...], wr_ref[...], preferred_element_type=jnp.float32)
         + bl_ref[...])
    o_ref[...] = jnp.maximum(t, 0.0) if relu else t


def _make_layer(relu):
    return pl.pallas_call(
        functools.partial(_layer_body, relu),
        grid=(NPAD // BLK,),
        in_specs=[
            pl.BlockSpec((NC, BLK, D), lambda i: (0, i, 0)),
            pl.BlockSpec((NC, NPAD), lambda i: (0, 0)),
            pl.BlockSpec((BLK, D), lambda i: (i, 0)),
            pl.BlockSpec((D, D), lambda i: (0, 0)),
            pl.BlockSpec((1, D), lambda i: (0, 0)),
            pl.BlockSpec((D, D), lambda i: (0, 0)),
        ],
        out_specs=pl.BlockSpec((BLK, D), lambda i: (i, 0)),
        out_shape=jax.ShapeDtypeStruct((NPAD, D), jnp.float32),
    )


_layer_relu = _make_layer(True)
_layer_lin = _make_layer(False)


def kernel(x, edge_index, Wl1, bl1, Wr1, Wl2, bl2, Wr2):
    pad = E_PAD - E
    src = jnp.concatenate(
        [edge_index[0], jnp.zeros((pad,), jnp.int32)]).reshape(
            NS, NCHUNK_TOT, CHUNK)
    dst = jnp.concatenate(
        [edge_index[1],
         DUMMY_LO + (jnp.arange(pad, dtype=jnp.int32) % (NPAD - N))]).reshape(
            NS, NCHUNK_TOT, CHUNK)
    p1, c1 = _seg_sum_cnt(x, src, dst)
    h = _layer_relu(p1, c1, x, Wl1, bl1.reshape(1, D), Wr1)
    (p2,) = _seg_sum(h, src, dst)
    out = _layer_lin(p2, c1, h, Wl2, bl2.reshape(1, D), Wr2)
    return out[:N]


# speedup vs baseline: 2.6260x; 2.2436x over previous
import functools

import jax
import jax.numpy as jnp
from jax import lax
from jax.experimental import pallas as pl
from jax.experimental.pallas import tpu as pltpu
from jax.experimental.pallas import tpu_sc as plsc

N = 10000
D = 128
HD = D // 2
E = 320000
NC = 2
NS = 16
CHUNK = 128
NCHUNK = 160
HCH = 32
NPIECE = NCHUNK // HCH
NBUF = 2
LAG = 1
NGRP = HCH // NBUF
E_PAD = NS * NCHUNK * CHUNK
NPAD = 10240
ROWS_PT = NPAD // NS
DUMMY_LO = N


def _seg_body(with_counts, x2_hbm, src_hbm, dst_hbm, psum_hbm, *rest):
    if with_counts:
        (pcnt_hbm, x_sh, acc_sh, cnt_sh, src_v, dst_v, rows, gsems, ssems,
         ones_v, czv) = rest
    else:
        (x_sh, acc_sh, src_v, dst_v, rows, gsems, ssems) = rest
        ones_v = czv = None
    c = lax.axis_index("c")
    s = lax.axis_index("s")

    zero16 = jnp.zeros((16,), jnp.float32)

    def _zrows(i, _):
        rows[0][i // (HD // 16), pl.ds((i % (HD // 16)) * 16, 16)] = zero16
        return 0
    lax.fori_loop(0, CHUNK * (HD // 16), _zrows, 0)

    if with_counts:
        one16 = jnp.ones((16,), jnp.float32)

        def _ones(i, _):
            ones_v[pl.ds(i * 16, 16)] = one16
            return 0
        lax.fori_loop(0, CHUNK // 16, _ones, 0)

        def _zc(i, _):
            czv[pl.ds(i * 16, 16)] = zero16
            return 0
        lax.fori_loop(0, ROWS_PT // 16, _zc, 0)

    pltpu.sync_copy(x2_hbm.at[c, pl.ds(s * ROWS_PT, ROWS_PT)],
                    x_sh.at[pl.ds(s * ROWS_PT, ROWS_PT)])

    def _zslab(k, _):
        pltpu.sync_copy(rows[0],
                        acc_sh.at[pl.ds(s * ROWS_PT + k * CHUNK, CHUNK)])
        return 0
    lax.fori_loop(0, ROWS_PT // CHUNK, _zslab, 0)
    if with_counts:
        pltpu.sync_copy(czv, cnt_sh.at[pl.ds(s * ROWS_PT, ROWS_PT)])
    plsc.subcore_barrier()

    def _start_gather(j, b):
        pltpu.async_copy(x_sh.at[src_v.at[j]], rows[b], gsems[b])

    def _wait_gather(j, b):
        pltpu.make_async_copy(x_sh.at[src_v.at[j]], rows[b],
                              gsems[b]).wait()

    def _start_scat(j, b):
        pltpu.async_copy(rows[b], acc_sh.at[dst_v.at[j]], ssems[b], add=True)

    def _wait_scat(j, b):
        pltpu.make_async_copy(rows[b], acc_sh.at[dst_v.at[j]],
                              ssems[b]).wait()

    def _counts(t):
        if with_counts:
            @pl.when((t + c) % 2 == 0)
            def _():
                pltpu.sync_copy(ones_v, cnt_sh.at[dst_v.at[t]], add=True)

    def _piece(p, _):
        base = pl.multiple_of(p * HCH, 8)
        pltpu.sync_copy(src_hbm.at[s, pl.ds(base, HCH)], src_v)
        pltpu.sync_copy(dst_hbm.at[s, pl.ds(base, HCH)], dst_v)
        for b in range(LAG):
            _start_gather(b, b)

        def _grp_body(gi, carry):
            for b in range(NBUF):
                t = gi * NBUF + b
                _wait_gather(t, b)
                _start_scat(t, b)
                _counts(t)
                pb = (b + LAG) % NBUF
                nxt = t + LAG
                if b < NBUF - LAG:
                    @pl.when(gi > 0)
                    def _():
                        _wait_scat(t - 1, pb)
                        _start_gather(nxt, pb)

                    @pl.when(gi == 0)
                    def _():
                        _start_gather(nxt, pb)
                else:
                    @pl.when(nxt < HCH)
                    def _():
                        _wait_scat(t - 1, pb)
                        _start_gather(nxt, pb)
            return carry
        lax.fori_loop(0, NGRP, _grp_body, 0)
        for b in range(NBUF):
            _wait_scat(HCH - NBUF + b, (HCH - NBUF + b) % NBUF)
        return 0
    lax.fori_loop(0, NPIECE, _piece, 0)
    plsc.subcore_barrier()

    pltpu.sync_copy(acc_sh.at[pl.ds(s * ROWS_PT, ROWS_PT)],
                    psum_hbm.at[c, pl.ds(s * ROWS_PT, ROWS_PT)])
    if with_counts:
        pltpu.sync_copy(cnt_sh.at[pl.ds(s * ROWS_PT, ROWS_PT)],
                        pcnt_hbm.at[c, pl.ds(s * ROWS_PT, ROWS_PT)])


def _make_seg_sum(with_counts):
    out_type = [jax.ShapeDtypeStruct((NC, NPAD, HD), jnp.float32)]
    scratch = [pltpu.VMEM_SHARED((NPAD, HD), jnp.float32),
               pltpu.VMEM_SHARED((NPAD, HD), jnp.float32)]
    if with_counts:
        out_type.append(jax.ShapeDtypeStruct((NC, NPAD), jnp.float32))
        scratch.append(pltpu.VMEM_SHARED((NPAD,), jnp.float32))
    scratch += [
        pltpu.VMEM((HCH, CHUNK), jnp.int32),
        pltpu.VMEM((HCH, CHUNK), jnp.int32),
        [pltpu.VMEM((CHUNK, HD), jnp.float32) for _ in range(NBUF)],
        [pltpu.SemaphoreType.DMA for _ in range(NBUF)],
        [pltpu.SemaphoreType.DMA for _ in range(NBUF)],
    ]
    if with_counts:
        scratch += [
            pltpu.VMEM((CHUNK,), jnp.float32),
            pltpu.VMEM((ROWS_PT,), jnp.float32),
        ]
    return pl.kernel(
        functools.partial(_seg_body, with_counts),
        out_type=tuple(out_type),
        mesh=plsc.VectorSubcoreMesh(core_axis_name="c", subcore_axis_name="s",
                                    num_cores=NC, num_subcores=NS),
        scratch_types=scratch,
    )


_seg_sum_cnt = _make_seg_sum(True)
_seg_sum = _make_seg_sum(False)


BLK = 1280


def _split_body(x_ref, o_ref):
    o_ref[0] = x_ref[:, :HD]
    o_ref[1] = x_ref[:, HD:]


_xsplit = pl.pallas_call(
    _split_body,
    grid=(NPAD // BLK,),
    in_specs=[pl.BlockSpec((BLK, D), lambda i: (i, 0))],
    out_specs=pl.BlockSpec((NC, BLK, HD), lambda i: (0, i, 0)),
    out_shape=jax.ShapeDtypeStruct((NC, NPAD, HD), jnp.float32),
)


def _layer_body(relu, p_ref, c_ref, x_ref, wl_ref, bl_ref, wr_ref, *outs):
    i = pl.program_id(0)
    cnt = c_ref[0, pl.ds(i * BLK, BLK)] + c_ref[1, pl.ds(i * BLK, BLK)]
    recip = 1.0 / jnp.maximum(cnt, 1.0)
    mean = jnp.concatenate([p_ref[0], p_ref[1]], axis=-1) * recip[:, None]
    t = (jnp.dot(mean, wl_ref[...], preferred_element_type=jnp.float32)
         + jnp.dot(x_ref[...], wr_ref[...], preferred_element_type=jnp.float32)
         + bl_ref[...])
    if relu:
        t = jnp.maximum(t, 0.0)
        outs[0][...] = t
        outs[1][0] = t[:, :HD]
        outs[1][1] = t[:, HD:]
    else:
        outs[0][...] = t


def _make_layer(relu):
    out_specs = [pl.BlockSpec((BLK, D), lambda i: (i, 0))]
    out_shape = [jax.ShapeDtypeStruct((NPAD, D), jnp.float32)]
    if relu:
        out_specs.append(pl.BlockSpec((NC, BLK, HD), lambda i: (0, i, 0)))
        out_shape.append(jax.ShapeDtypeStruct((NC, NPAD, HD), jnp.float32))
    return pl.pallas_call(
        functools.partial(_layer_body, relu),
        grid=(NPAD // BLK,),
        in_specs=[
            pl.BlockSpec((NC, BLK, HD), lambda i: (0, i, 0)),
            pl.BlockSpec((NC, NPAD), lambda i: (0, 0)),
            pl.BlockSpec((BLK, D), lambda i: (i, 0)),
            pl.BlockSpec((D, D), lambda i: (0, 0)),
            pl.BlockSpec((1, D), lambda i: (0, 0)),
            pl.BlockSpec((D, D), lambda i: (0, 0)),
        ],
        out_specs=out_specs,
        out_shape=out_shape,
    )


_layer_relu = _make_layer(True)
_layer_lin = _make_layer(False)


def kernel(x, edge_index, Wl1, bl1, Wr1, Wl2, bl2, Wr2):
    pad = E_PAD - E
    src = jnp.concatenate(
        [edge_index[0], jnp.zeros((pad,), jnp.int32)]).reshape(
            NS, NCHUNK, CHUNK)
    dst = jnp.concatenate(
        [edge_index[1],
         DUMMY_LO + (jnp.arange(pad, dtype=jnp.int32) % (NPAD - N))]).reshape(
            NS, NCHUNK, CHUNK)
    x2 = _xsplit(x)
    p1, c1 = _seg_sum_cnt(x2, src, dst)
    h, h2 = _layer_relu(p1, c1, x, Wl1, bl1.reshape(1, D), Wr1)
    (p2,) = _seg_sum(h2, src, dst)
    (out,) = _layer_lin(p2, c1, h, Wl2, bl2.reshape(1, D), Wr2)
    return out[:N]
